# baseline (speedup 1.0000x reference)
import dataclasses

import jax
import jax.numpy as jnp
from jax import lax
from jax.experimental import pallas as pl
from jax.experimental.pallas import tpu as pltpu
from jax.experimental.pallas import tpu_sc as plsc

SVC_NUM = 1000
INSTANCE_NUM = 6000
NODE_NUM = 3000
D = 128
E_SVC = 32000
E_IN = 192000
E_NI = 192000

NUM_CORES = 2
NUM_SUBCORES = 16
NUM_TILES = NUM_CORES * NUM_SUBCORES

OD_SVC = 0
ID_SVC = 1008
OD_INST = 2016
ID_NODE = 8016
OD_NODE = 11024
ID_INST = 14032
NB = 20480

E_SVC_PAD = 33792
EH = 2 * E_SVC_PAD + 2 * E_IN + 2 * E_NI
H_PER_TILE = EH // NUM_TILES


TBL_SVC = 0
TBL_INST = SVC_NUM
TBL_NODE = SVC_NUM + INSTANCE_NUM
N_SRC = SVC_NUM + INSTANCE_NUM + NODE_NUM

AGG_NODE = 0
AGG_SVC = 3008
AGG_INST = 0
AGG_SVC1 = 6000
NAGG = 7040
STRIPE = NAGG // NUM_SUBCORES

KE = 96
NSTEP = 138
NBUF = 3
JUNK_ROW = 7008

_MESH = lambda: plsc.VectorSubcoreMesh(core_axis_name="c", subcore_axis_name="s")


def _compiler_params():
    cp = pltpu.CompilerParams()
    if "needs_layout_passes" in pltpu.CompilerParams.__dataclass_fields__:
        cp = dataclasses.replace(cp, needs_layout_passes=False)
    return cp


def _hist_call(hist_idx):
    @pl.kernel(
        out_type=jax.ShapeDtypeStruct((NUM_TILES, NB), jnp.float32),
        mesh=_MESH(),
        scratch_types=[
            pltpu.VMEM((NB,), jnp.float32),
            pltpu.VMEM((H_PER_TILE // 128, 128), jnp.int32),
            pltpu.SemaphoreType.DMA,
        ],
        compiler_params=_compiler_params(),
    )
    def hist_kernel(idx_hbm, out_hbm, bins_v, idx_v, isem):
        c = lax.axis_index("c")
        s = lax.axis_index("s")
        wid = c * NUM_SUBCORES + s
        zeros16 = jnp.zeros((16,), jnp.float32)
        ones16 = jnp.ones((16,), jnp.float32)

        pltpu.async_copy(idx_hbm.at[wid], idx_v, isem)

        @pl.loop(0, NB, step=16)
        def _(i):
            bins_v[pl.ds(i, 16)] = zeros16

        pltpu.make_async_copy(idx_hbm.at[wid], idx_v, isem).wait()

        @pl.loop(0, H_PER_TILE // 128)
        def _(r):
            @pl.loop(0, 128, step=16)
            def _(j):
                idx = idx_v[r, pl.ds(j, 16)]
                plsc.addupdate_scatter(bins_v, [idx], ones16)

        pltpu.sync_copy(bins_v, out_hbm.at[wid])

    return hist_kernel(hist_idx.reshape(NUM_TILES, H_PER_TILE // 128, 128))


def _agg_call(src_all, dst_all, xs, zeros_stripe):
    @pl.kernel(
        out_type=jax.ShapeDtypeStruct((NUM_CORES, NAGG, D), jnp.float32),
        mesh=_MESH(),
        scratch_types=(
            [pltpu.VMEM((KE, D), jnp.float32) for _ in range(NBUF)]
            + [
                pltpu.VMEM((NSTEP, KE), jnp.int32),
                pltpu.VMEM((NSTEP, KE), jnp.int32),
                pltpu.VMEM_SHARED((NAGG, D), jnp.float32),
            ]
            + [pltpu.SemaphoreType.DMA for _ in range(NBUF + 1)]
        ),
        compiler_params=_compiler_params(),
    )
    def agg_kernel(src_hbm, dst_hbm, xs_hbm, z_hbm, out_hbm,
                   r0, r1, r2, src_v, dst_v, agg_sh,
                   g0, g1, g2, isem):
        rows_v = (r0, r1, r2)
        gsem = (g0, g1, g2)
        c = lax.axis_index("c")
        s = lax.axis_index("s")

        pltpu.async_copy(src_hbm.at[c, s], src_v, isem)
        pltpu.sync_copy(z_hbm, agg_sh.at[pl.ds(s * STRIPE, STRIPE)])
        pltpu.async_copy(dst_hbm.at[c, s], dst_v, isem)
        pltpu.make_async_copy(src_hbm.at[c, s], src_v, isem).wait()
        pltpu.make_async_copy(dst_hbm.at[c, s], dst_v, isem).wait()
        plsc.subcore_barrier()

        for b in range(NBUF):
            pltpu.async_copy(xs_hbm.at[src_v.at[b]], rows_v[b], gsem[b])

        @pl.loop(0, NSTEP, step=NBUF)
        def _(g):
            for b in range(NBUF):
                chunk = g + b
                pltpu.make_async_copy(
                    xs_hbm.at[src_v.at[chunk]], rows_v[b], gsem[b]).wait()
                pltpu.sync_copy(
                    rows_v[b], agg_sh.at[dst_v.at[chunk]], add=True)

                @pl.when(chunk + NBUF < NSTEP)
                def _():
                    pltpu.async_copy(
                        xs_hbm.at[src_v.at[chunk + NBUF]], rows_v[b], gsem[b])

        plsc.subcore_barrier()
        pltpu.sync_copy(
            agg_sh.at[pl.ds(s * STRIPE, STRIPE)],
            out_hbm.at[c, pl.ds(s * STRIPE, STRIPE)],
        )

    return agg_kernel(src_all, dst_all, xs, zeros_stripe)


def _scale_src_call(hist32, big_x):
    def body(hist_ref, x_ref, xs_ref, rs_ref):
        h = jnp.sum(hist_ref[...], axis=0)
        rs = lax.rsqrt(jnp.maximum(h, 1.0))
        rs_ref[...] = rs
        rsf = rs.reshape(-1)
        scale = jnp.concatenate([
            rsf[OD_SVC:OD_SVC + SVC_NUM],
            rsf[OD_INST:OD_INST + INSTANCE_NUM],
            rsf[OD_NODE:OD_NODE + NODE_NUM],
        ])
        xs_ref[...] = x_ref[...] * scale[:, None]

    return pl.pallas_call(
        body,
        out_shape=(
            jax.ShapeDtypeStruct((N_SRC, D), jnp.float32),
            jax.ShapeDtypeStruct((NB // 128, 128), jnp.float32),
        ),
    )(hist32, big_x)


def _finish_call(aggp, rs2, W_inst, b_inst, W_node, b_node, W_svc, b_svc):
    def body(a_ref, rs_ref, wi_ref, bi_ref, wn_ref, bn_ref, ws_ref, bs_ref,
             o_ref):
        rsf = rs_ref[...].reshape(-1)

        def rel(agg, seg, n, w, b):
            a = agg * rsf[seg:seg + n][:, None]
            z = jnp.dot(a, w, preferred_element_type=jnp.float32) + b[None, :]
            return jnp.maximum(z, 0.0) + 0.01 * jnp.minimum(z, 0.0)

        o_ref[0:NODE_NUM] = rel(
            a_ref[0, AGG_NODE:AGG_NODE + NODE_NUM], ID_NODE, NODE_NUM,
            wi_ref[...], bi_ref[...])
        o_ref[NODE_NUM:NODE_NUM + INSTANCE_NUM] = rel(
            a_ref[1, AGG_INST:AGG_INST + INSTANCE_NUM], ID_INST, INSTANCE_NUM,
            wn_ref[...], bn_ref[...])
        o_ref[NODE_NUM + INSTANCE_NUM:] = rel(
            a_ref[0, AGG_SVC:AGG_SVC + SVC_NUM]
            + a_ref[1, AGG_SVC1:AGG_SVC1 + SVC_NUM], ID_SVC, SVC_NUM,
            ws_ref[...], bs_ref[...])

    return pl.pallas_call(
        body,
        out_shape=jax.ShapeDtypeStruct((NODE_NUM + INSTANCE_NUM + SVC_NUM, D),
                                       jnp.float32),
    )(aggp, rs2, W_inst, b_inst, W_node, b_node, W_svc, b_svc)


def kernel(svc_feat, instance_feat, node_feat, svc_edge_index,
           instance_node_edge_index, node_instance_edge_index,
           W_svc, b_svc, W_inst, b_inst, W_node, b_node):
    svc_e = svc_edge_index.astype(jnp.int32)
    in_e = instance_node_edge_index.astype(jnp.int32)
    ni_e = node_instance_edge_index.astype(jnp.int32)

    pad = jnp.full((E_SVC_PAD - E_SVC,), SVC_NUM, jnp.int32)
    hist_idx = jnp.concatenate([
        svc_e[0] + OD_SVC, pad + OD_SVC,
        svc_e[1] + ID_SVC, pad + ID_SVC,
        in_e[0] + OD_INST, in_e[1] + ID_NODE,
        ni_e[0] + OD_NODE, ni_e[1] + ID_INST,
    ])

    half = E_SVC // 2
    n_pad = NUM_SUBCORES * NSTEP * KE - (E_IN + half)
    zpad = jnp.zeros((n_pad,), jnp.int32)
    jpad = jnp.full((n_pad,), JUNK_ROW, jnp.int32)
    src_c0 = jnp.concatenate([
        in_e[0] + TBL_INST, svc_e[0, :half] + TBL_SVC, zpad])
    dst_c0 = jnp.concatenate([
        in_e[1] + AGG_NODE, svc_e[1, :half] + AGG_SVC, jpad])
    src_c1 = jnp.concatenate([
        ni_e[0] + TBL_NODE, svc_e[0, half:] + TBL_SVC, zpad])
    dst_c1 = jnp.concatenate([
        ni_e[1] + AGG_INST, svc_e[1, half:] + AGG_SVC1, jpad])

    def _slab(c0, c1):
        return jnp.stack([c0.reshape(NUM_SUBCORES, NSTEP, KE),
                          c1.reshape(NUM_SUBCORES, NSTEP, KE)])

    src_all = _slab(src_c0, src_c1)
    dst_all = _slab(dst_c0, dst_c1)

    big_x = jnp.concatenate([svc_feat, instance_feat, node_feat], axis=0)

    hist32 = _hist_call(hist_idx).reshape(NUM_TILES, NB // 128, 128)
    xs, rs2 = _scale_src_call(hist32, big_x)

    zeros_stripe = jnp.zeros((STRIPE, D), jnp.float32)
    aggp = _agg_call(src_all, dst_all, xs, zeros_stripe)

    return _finish_call(aggp, rs2, W_inst, b_inst, W_node, b_node,
                        W_svc, b_svc)


# speedup vs baseline: 1.8503x; 1.8503x over previous
import dataclasses

import jax
import jax.numpy as jnp
from jax import lax
from jax.experimental import pallas as pl
from jax.experimental.pallas import tpu as pltpu
from jax.experimental.pallas import tpu_sc as plsc

SVC_NUM = 1000
INSTANCE_NUM = 6000
NODE_NUM = 3000
D = 128
E_SVC = 32000
E_IN = 192000
E_NI = 192000

NUM_CORES = 2
NUM_SUBCORES = 16
NUM_TILES = NUM_CORES * NUM_SUBCORES

OD_SVC = 0
ID_SVC = 1008
OD_INST = 2016
ID_NODE = 8016
OD_NODE = 11024
ID_INST = 14032
NB = 20480

E_SVC_PAD = 33792
EH = 2 * E_SVC_PAD + 2 * E_IN + 2 * E_NI
H_PER_TILE = EH // NUM_TILES


TBL_SVC = 0
TBL_INST = SVC_NUM
TBL_NODE = SVC_NUM + INSTANCE_NUM
N_SRC = SVC_NUM + INSTANCE_NUM + NODE_NUM

AGG_NODE = 0
AGG_SVC = 3008
AGG_INST = 0
NAGG = 6016
STRIPE = NAGG // NUM_SUBCORES

KE = 128
NSTEP0 = 110
NSTEP1 = 94
NBUF = 3
LOOP_HI = 111
JUNK_ROW = 6000

_MESH = lambda: plsc.VectorSubcoreMesh(core_axis_name="c", subcore_axis_name="s")


def _compiler_params():
    cp = pltpu.CompilerParams()
    if "needs_layout_passes" in pltpu.CompilerParams.__dataclass_fields__:
        cp = dataclasses.replace(cp, needs_layout_passes=False)
    return cp


def _hist_call(hist_idx):
    @pl.kernel(
        out_type=jax.ShapeDtypeStruct((NUM_TILES, NB), jnp.float32),
        mesh=_MESH(),
        scratch_types=[
            pltpu.VMEM((NB,), jnp.float32),
            pltpu.VMEM((H_PER_TILE // 128, 128), jnp.int32),
            pltpu.SemaphoreType.DMA,
        ],
        compiler_params=_compiler_params(),
    )
    def hist_kernel(idx_hbm, out_hbm, bins_v, idx_v, isem):
        c = lax.axis_index("c")
        s = lax.axis_index("s")
        wid = c * NUM_SUBCORES + s
        zeros16 = jnp.zeros((16,), jnp.float32)
        ones16 = jnp.ones((16,), jnp.float32)

        pltpu.async_copy(idx_hbm.at[wid], idx_v, isem)

        @pl.loop(0, NB, step=16)
        def _(i):
            bins_v[pl.ds(i, 16)] = zeros16

        pltpu.make_async_copy(idx_hbm.at[wid], idx_v, isem).wait()

        @pl.loop(0, H_PER_TILE // 128)
        def _(r):
            @pl.loop(0, 128, step=16)
            def _(j):
                idx = idx_v[r, pl.ds(j, 16)]
                plsc.addupdate_scatter(bins_v, [idx], ones16)

        pltpu.sync_copy(bins_v, out_hbm.at[wid])

    return hist_kernel(hist_idx.reshape(NUM_TILES, H_PER_TILE // 128, 128))


def _agg_call(src_all, dst_all, xs, zeros_stripe):
    @pl.kernel(
        out_type=jax.ShapeDtypeStruct((NUM_CORES, NAGG, D), jnp.float32),
        mesh=_MESH(),
        scratch_types=(
            [pltpu.VMEM((KE, D), jnp.float32) for _ in range(NBUF)]
            + [
                pltpu.VMEM((NSTEP0, KE), jnp.int32),
                pltpu.VMEM((NSTEP0, KE), jnp.int32),
                pltpu.VMEM_SHARED((NAGG, D), jnp.float32),
            ]
            + [pltpu.SemaphoreType.DMA for _ in range(NBUF + 1)]
        ),
        compiler_params=_compiler_params(),
    )
    def agg_kernel(src_hbm, dst_hbm, xs_hbm, z_hbm, out_hbm,
                   r0, r1, r2, src_v, dst_v, agg_sh,
                   g0, g1, g2, isem):
        rows_v = (r0, r1, r2)
        gsem = (g0, g1, g2)
        c = lax.axis_index("c")
        s = lax.axis_index("s")
        nstep = jnp.where(c == 0, NSTEP0, NSTEP1)

        pltpu.async_copy(src_hbm.at[c, s], src_v, isem)
        pltpu.sync_copy(z_hbm, agg_sh.at[pl.ds(s * STRIPE, STRIPE)])
        pltpu.async_copy(dst_hbm.at[c, s], dst_v, isem)
        pltpu.make_async_copy(src_hbm.at[c, s], src_v, isem).wait()
        pltpu.make_async_copy(dst_hbm.at[c, s], dst_v, isem).wait()
        plsc.subcore_barrier()

        for b in range(NBUF):
            pltpu.async_copy(xs_hbm.at[src_v.at[b]], rows_v[b], gsem[b])

        @pl.loop(0, LOOP_HI, step=NBUF)
        def _(g):
            for b in range(NBUF):
                chunk = g + b

                @pl.when(chunk < nstep)
                def _():
                    pltpu.make_async_copy(
                        xs_hbm.at[src_v.at[chunk]], rows_v[b], gsem[b]).wait()
                    pltpu.sync_copy(
                        rows_v[b], agg_sh.at[dst_v.at[chunk]], add=True)

                    @pl.when(chunk + NBUF < nstep)
                    def _():
                        pltpu.async_copy(
                            xs_hbm.at[src_v.at[chunk + NBUF]], rows_v[b],
                            gsem[b])

        plsc.subcore_barrier()
        pltpu.sync_copy(
            agg_sh.at[pl.ds(s * STRIPE, STRIPE)],
            out_hbm.at[c, pl.ds(s * STRIPE, STRIPE)],
        )

    return agg_kernel(src_all, dst_all, xs, zeros_stripe)


def _scale_src_call(hist32, big_x):
    def body(hist_ref, x_ref, xs_ref, rs_ref):
        h = jnp.sum(hist_ref[...], axis=0)
        rs = lax.rsqrt(jnp.maximum(h, 1.0))
        rs_ref[...] = rs
        rsf = rs.reshape(-1)
        scale = jnp.concatenate([
            rsf[OD_SVC:OD_SVC + SVC_NUM],
            rsf[OD_INST:OD_INST + INSTANCE_NUM],
            rsf[OD_NODE:OD_NODE + NODE_NUM],
        ])
        xs_ref[...] = x_ref[...] * scale[:, None]

    return pl.pallas_call(
        body,
        out_shape=(
            jax.ShapeDtypeStruct((N_SRC, D), jnp.float32),
            jax.ShapeDtypeStruct((NB // 128, 128), jnp.float32),
        ),
    )(hist32, big_x)


def _finish_call(aggp, rs2, W_inst, b_inst, W_node, b_node, W_svc, b_svc):
    def body(a_ref, rs_ref, wi_ref, bi_ref, wn_ref, bn_ref, ws_ref, bs_ref,
             o_ref):
        rsf = rs_ref[...].reshape(-1)

        def rel(agg, seg, n, w, b):
            a = agg * rsf[seg:seg + n][:, None]
            z = jnp.dot(a, w, preferred_element_type=jnp.float32) + b[None, :]
            return jnp.maximum(z, 0.0) + 0.01 * jnp.minimum(z, 0.0)

        o_ref[0:NODE_NUM] = rel(
            a_ref[0, AGG_NODE:AGG_NODE + NODE_NUM], ID_NODE, NODE_NUM,
            wi_ref[...], bi_ref[...])
        o_ref[NODE_NUM:NODE_NUM + INSTANCE_NUM] = rel(
            a_ref[1, AGG_INST:AGG_INST + INSTANCE_NUM], ID_INST, INSTANCE_NUM,
            wn_ref[...], bn_ref[...])
        o_ref[NODE_NUM + INSTANCE_NUM:] = rel(
            a_ref[0, AGG_SVC:AGG_SVC + SVC_NUM], ID_SVC, SVC_NUM,
            ws_ref[...], bs_ref[...])

    return pl.pallas_call(
        body,
        out_shape=jax.ShapeDtypeStruct((NODE_NUM + INSTANCE_NUM + SVC_NUM, D),
                                       jnp.float32),
    )(aggp, rs2, W_inst, b_inst, W_node, b_node, W_svc, b_svc)


def kernel(svc_feat, instance_feat, node_feat, svc_edge_index,
           instance_node_edge_index, node_instance_edge_index,
           W_svc, b_svc, W_inst, b_inst, W_node, b_node):
    svc_e = svc_edge_index.astype(jnp.int32)
    in_e = instance_node_edge_index.astype(jnp.int32)
    ni_e = node_instance_edge_index.astype(jnp.int32)

    pad = jnp.full((E_SVC_PAD - E_SVC,), SVC_NUM, jnp.int32)
    hist_idx = jnp.concatenate([
        svc_e[0] + OD_SVC, pad + OD_SVC,
        svc_e[1] + ID_SVC, pad + ID_SVC,
        in_e[0] + OD_INST, in_e[1] + ID_NODE,
        ni_e[0] + OD_NODE, ni_e[1] + ID_INST,
    ])

    n_pad0 = NUM_SUBCORES * NSTEP0 * KE - (E_SVC + E_IN)
    n_pad1 = NUM_SUBCORES * NSTEP1 * KE - E_NI
    src_c0 = jnp.concatenate([
        svc_e[0] + TBL_SVC, in_e[0] + TBL_INST,
        jnp.zeros((n_pad0,), jnp.int32)])
    dst_c0 = jnp.concatenate([
        svc_e[1] + AGG_SVC, in_e[1] + AGG_NODE,
        jnp.full((n_pad0,), JUNK_ROW, jnp.int32)])
    src_c1 = jnp.concatenate([
        ni_e[0] + TBL_NODE, jnp.zeros((n_pad1,), jnp.int32)])
    dst_c1 = jnp.concatenate([
        ni_e[1] + AGG_INST, jnp.full((n_pad1,), JUNK_ROW, jnp.int32)])
    pad_e = jnp.zeros((NUM_SUBCORES, (NSTEP0 - NSTEP1), KE), jnp.int32)

    def _slab(c0, c1):
        c0 = c0.reshape(NUM_SUBCORES, NSTEP0, KE)
        c1 = jnp.concatenate(
            [c1.reshape(NUM_SUBCORES, NSTEP1, KE), pad_e], axis=1)
        return jnp.stack([c0, c1])

    src_all = _slab(src_c0, src_c1)
    dst_all = _slab(dst_c0, dst_c1)

    big_x = jnp.concatenate([svc_feat, instance_feat, node_feat], axis=0)

    hist32 = _hist_call(hist_idx).reshape(NUM_TILES, NB // 128, 128)
    xs, rs2 = _scale_src_call(hist32, big_x)

    zeros_stripe = jnp.zeros((STRIPE, D), jnp.float32)
    aggp = _agg_call(src_all, dst_all, xs, zeros_stripe)

    return _finish_call(aggp, rs2, W_inst, b_inst, W_node, b_node,
                        W_svc, b_svc)
